# baseline (speedup 1.0000x reference)
import jax
import jax.numpy as jnp
from jax import lax
from jax.experimental import pallas as pl
from jax.experimental.pallas import tpu as pltpu
from jax.experimental.pallas import tpu_sc as plsc

N = 10000
NNZ = 320000
FIN = 128
D = 2
H = 64
DH = D * H
NCLS = 40
NE = 10000
ALPHA = 0.5

NCORE = 2
NSUB = 16
NWORK = NCORE * NSUB
PS = NNZ // NWORK
CH = 80
NCHUNK = PS // CH
RZ = 1000
NZSUB = N // RZ

_SC_MESH = dict(core_axis_name="c", subcore_axis_name="s")


def _wid():
    return lax.axis_index("s") * NCORE + lax.axis_index("c")


def _zero_acc(z_hbm, acc, sid):
    @pl.when(sid < NZSUB)
    def _():
        pltpu.sync_copy(z_hbm, acc.at[pl.ds(sid * RZ, RZ)])


def _writeout(acc, out_hbm, cid, sid):
    @pl.when(sid < NZSUB)
    def _():
        pltpu.sync_copy(acc.at[pl.ds(sid * RZ, RZ)],
                        out_hbm.at[cid, pl.ds(sid * RZ, RZ)])


def _sc_scaled_body(t_hbm, s_hbm, d_hbm, hd_hbm, a_hbm, b_hbm, flag_hbm,
                    z128, z16, out_hbm, cnt_hbm, hdo_hbm,
                    sidx0, didx0, hdbuf0, sidx1, didx1, hdbuf1,
                    sidx2, didx2, hdbuf2, sidx3, didx3, hdbuf3,
                    rows0, rows1, arow0, brow0, arow1, brow1, flagbuf,
                    seml0, seml1, seml2, seml3, semg0, semg1, sema0, sema1,
                    acc, cacc):
    cid = lax.axis_index("c")
    sid = lax.axis_index("s")
    wid = _wid()
    _zero_acc(z128, acc, sid)

    @pl.when(sid < NZSUB)
    def _():
        pltpu.sync_copy(z16, cacc.at[pl.ds(sid * RZ, RZ)])
    plsc.subcore_barrier()

    base = wid * PS
    pltpu.sync_copy(flag_hbm, flagbuf)
    mode = flagbuf[pl.ds(0, 16)][0]
    do_scale = mode >= 1
    fused = mode == 2

    sets = ((sidx0, didx0, hdbuf0, seml0), (sidx1, didx1, hdbuf1, seml1),
            (sidx2, didx2, hdbuf2, seml2), (sidx3, didx3, hdbuf3, seml3))
    rbufs = ((rows0, semg0, sema0, arow0, brow0),
             (rows1, semg1, sema1, arow1, brow1))
    lane0 = jnp.zeros((16, 1), jnp.int32)
    lane1 = jnp.ones((16, 1), jnp.int32)
    gdn = lax.GatherDimensionNumbers(offset_dims=(), collapsed_slice_dims=(0,),
                                     start_index_map=(0,))

    def bcast(hv, lanes):
        return lax.gather(hv, lanes, gdn, (1,),
                          mode=lax.GatherScatterMode.PROMISE_IN_BOUNDS)

    def issue_linear(j, q):
        off = base + j * CH
        pltpu.async_copy(s_hbm.at[pl.ds(off, CH)], sets[q][0], sets[q][3])
        pltpu.async_copy(d_hbm.at[pl.ds(off, CH)], sets[q][1], sets[q][3])

        @pl.when(jnp.logical_not(fused))
        def _():
            pltpu.async_copy(hd_hbm.at[pl.ds(off, CH)], sets[q][2], sets[q][3])

    def wait_linear(q):
        pltpu.make_async_copy(s_hbm.at[pl.ds(0, CH)], sets[q][0], sets[q][3]).wait()
        pltpu.make_async_copy(d_hbm.at[pl.ds(0, CH)], sets[q][1], sets[q][3]).wait()

        @pl.when(jnp.logical_not(fused))
        def _():
            pltpu.make_async_copy(hd_hbm.at[pl.ds(0, CH)], sets[q][2],
                                  sets[q][3]).wait()

    def issue_gathers(r, q):
        pltpu.async_copy(t_hbm.at[sets[q][0]], rbufs[r][0], rbufs[r][1])

        @pl.when(fused)
        def _():
            pltpu.async_copy(a_hbm.at[sets[q][0]], rbufs[r][3], rbufs[r][1])
            pltpu.async_copy(b_hbm.at[sets[q][1]], rbufs[r][4], rbufs[r][1])

    def wait_gathers(r, q):
        pltpu.make_async_copy(t_hbm.at[sets[q][0]], rbufs[r][0],
                              rbufs[r][1]).wait()

        @pl.when(fused)
        def _():
            pltpu.make_async_copy(a_hbm.at[sets[q][0]], rbufs[r][3],
                                  rbufs[r][1]).wait()
            pltpu.make_async_copy(b_hbm.at[sets[q][1]], rbufs[r][4],
                                  rbufs[r][1]).wait()

    def issue_scatters(r, q, j):
        pltpu.async_copy(rbufs[r][0], acc.at[sets[q][1]], rbufs[r][2],
                         add=True)
        pltpu.async_copy(sets[q][2], cacc.at[sets[q][1]], rbufs[r][2],
                         add=True)

        @pl.when(fused)
        def _():
            off = base + j * CH
            pltpu.async_copy(sets[q][2], hdo_hbm.at[pl.ds(off, CH)],
                             rbufs[r][2])

    def wait_scatter(r, q):
        pltpu.make_async_copy(rbufs[r][0], acc.at[sets[q][1]],
                              rbufs[r][2]).wait()
        pltpu.make_async_copy(sets[q][2], cacc.at[sets[q][1]],
                              rbufs[r][2]).wait()

        @pl.when(fused)
        def _():
            pltpu.make_async_copy(sets[q][2], hdo_hbm.at[pl.ds(0, CH)],
                                  rbufs[r][2]).wait()

    def compute(r, q):
        rows, hdbuf = rbufs[r][0], sets[q][2]
        arow, brow = rbufs[r][3], rbufs[r][4]

        @pl.when(fused)
        def _():
            @plsc.parallel_loop(0, CH, 1, unroll=2)
            def _(i):
                y = arow[i, :] + brow[i, :]
                u = jnp.exp(jnp.float32(-2.0) * jnp.abs(y))
                t = jnp.sign(y) * (jnp.float32(1.0) - u) / (jnp.float32(1.0) + u)
                hdbuf[i, :] = t

        @pl.when(do_scale)
        def _():
            @plsc.parallel_loop(0, CH, 1, unroll=2)
            def _(i):
                hv = hdbuf[i, :]
                s0 = bcast(hv, lane0)
                s1 = bcast(hv, lane1)
                for k in range(8):
                    sv = s0 if k < 4 else s1
                    rows[i, pl.ds(k * 16, 16)] = rows[i, pl.ds(k * 16, 16)] * sv

    issue_linear(0, 0)
    wait_linear(0)
    issue_gathers(0, 0)
    issue_linear(1, 1)

    def step(jj, _):
        for t in range(4):
            r, nr, q = t % 2, 1 - t % 2, t
            j = jj * 4 + t
            wait_gathers(r, q)
            @pl.when(j >= 1)
            def _():
                wait_scatter(nr, (t + 3) % 4)
            wait_linear((t + 1) % 4)
            issue_gathers(nr, (t + 1) % 4)
            @pl.when(j + 2 < NCHUNK)
            def _():
                issue_linear(j + 2, (t + 2) % 4)
            compute(r, q)
            issue_scatters(r, q, j)
        return 0
    lax.fori_loop(0, NCHUNK // 4, step, 0)

    wait_scatter(1, 3)
    wait_gathers(0, 0)
    compute(0, 0)
    pltpu.sync_copy(rbufs[0][0], acc.at[sets[0][1]], add=True)
    pltpu.sync_copy(sets[0][2], cacc.at[sets[0][1]], add=True)

    @pl.when(fused)
    def _():
        off = base + (NCHUNK - 1) * CH
        pltpu.sync_copy(sets[0][2], hdo_hbm.at[pl.ds(off, CH)])

    plsc.subcore_barrier()
    _writeout(acc, out_hbm, cid, sid)

    @pl.when(sid < NZSUB)
    def _():
        pltpu.sync_copy(cacc.at[pl.ds(sid * RZ, RZ)],
                        cnt_hbm.at[cid, pl.ds(sid * RZ, RZ)])


def _sc_scaled(table, src, dst, hd, a_pad, b_pad, flag, z128, z16):
    return pl.kernel(
        _sc_scaled_body,
        out_type=[jax.ShapeDtypeStruct((NCORE, N, FIN), jnp.float32),
                  jax.ShapeDtypeStruct((NCORE, N, 16), jnp.float32),
                  jax.ShapeDtypeStruct((NNZ, 16), jnp.float32)],
        mesh=plsc.VectorSubcoreMesh(**_SC_MESH),
        compiler_params=pltpu.CompilerParams(use_tc_tiling_on_sc=False),
        scratch_types=(
            [pltpu.VMEM((CH,), jnp.int32), pltpu.VMEM((CH,), jnp.int32),
             pltpu.VMEM((CH, 16), jnp.float32)] * 4 +
            [pltpu.VMEM((CH, FIN), jnp.float32)] * 2 +
            [pltpu.VMEM((CH, 16), jnp.float32)] * 4 +
            [pltpu.VMEM((16,), jnp.int32)] +
            [pltpu.SemaphoreType.DMA] * 8 +
            [pltpu.VMEM_SHARED((N, FIN), jnp.float32),
             pltpu.VMEM_SHARED((N, 16), jnp.float32)]
        ),
    )(table, src, dst, hd, a_pad, b_pad, flag, z128, z16)


def _sc_sheaf_body(a_hbm, b_hbm, v_hbm, e_hbm, hd_hbm,
                   sidx0, didx0, arow0, brow0, trow0,
                   sidx1, didx1, arow1, brow1, trow1,
                   seml0, seml1, semg0, semg1, semw0, semw1):
    wid = _wid()
    base = wid * PS
    bufs = ((sidx0, didx0, arow0, brow0, trow0, seml0, semg0, semw0),
            (sidx1, didx1, arow1, brow1, trow1, seml1, semg1, semw1))

    def issue_linear(j, b):
        off = base + j * CH
        pltpu.async_copy(v_hbm.at[pl.ds(off, CH)], b[0], b[5])
        pltpu.async_copy(e_hbm.at[pl.ds(off, CH)], b[1], b[5])

    def wait_linear(b):
        pltpu.make_async_copy(v_hbm.at[pl.ds(0, CH)], b[0], b[5]).wait()
        pltpu.make_async_copy(e_hbm.at[pl.ds(0, CH)], b[1], b[5]).wait()

    def issue_gathers(b):
        pltpu.async_copy(a_hbm.at[b[0]], b[2], b[6])
        pltpu.async_copy(b_hbm.at[b[1]], b[3], b[6])

    def wait_gathers(b):
        pltpu.make_async_copy(a_hbm.at[b[0]], b[2], b[6]).wait()
        pltpu.make_async_copy(b_hbm.at[b[1]], b[3], b[6]).wait()

    def compute(b):
        arow, brow, trow = b[2], b[3], b[4]

        @plsc.parallel_loop(0, CH, 1, unroll=4)
        def _(i):
            y = arow[i, :] + brow[i, :]
            u = jnp.exp(jnp.float32(-2.0) * jnp.abs(y))
            t = jnp.sign(y) * (jnp.float32(1.0) - u) / (jnp.float32(1.0) + u)
            trow[i, :] = t

    issue_linear(0, bufs[0])
    wait_linear(bufs[0])
    issue_gathers(bufs[0])
    issue_linear(1, bufs[1])

    def step(jj, _):
        for bsel in range(2):
            b = bufs[bsel]
            nb = bufs[1 - bsel]
            j = jj * 2 + bsel
            wait_gathers(b)

            @pl.when(j + 1 < NCHUNK)
            def _():
                wait_linear(nb)
                issue_gathers(nb)

            @pl.when(j >= 2)
            def _():
                pltpu.make_async_copy(
                    b[4], hd_hbm.at[pl.ds(0, CH)], b[7]).wait()

            @pl.when(j + 2 < NCHUNK)
            def _():
                issue_linear(j + 2, b)
            compute(b)
            off = base + j * CH
            pltpu.async_copy(b[4], hd_hbm.at[pl.ds(off, CH)], b[7])
        return 0
    lax.fori_loop(0, NCHUNK // 2, step, 0)

    if NCHUNK % 2:
        b = bufs[(NCHUNK - 1) % 2]
        wait_gathers(b)
        pltpu.make_async_copy(b[4], hd_hbm.at[pl.ds(0, CH)], b[7]).wait()
        compute(b)
        off = base + (NCHUNK - 1) * CH
        pltpu.async_copy(b[4], hd_hbm.at[pl.ds(off, CH)], b[7])

    pltpu.make_async_copy(
        bufs[(NCHUNK - 1) % 2][4], hd_hbm.at[pl.ds(0, CH)],
        bufs[(NCHUNK - 1) % 2][7]).wait()
    pltpu.make_async_copy(
        bufs[(NCHUNK - 2) % 2][4], hd_hbm.at[pl.ds(0, CH)],
        bufs[(NCHUNK - 2) % 2][7]).wait()


def _sc_sheaf(a_pad, b_pad, vidx, eidx):
    return pl.kernel(
        _sc_sheaf_body,
        out_type=jax.ShapeDtypeStruct((NNZ, 16), jnp.float32),
        mesh=plsc.VectorSubcoreMesh(**_SC_MESH),
        compiler_params=pltpu.CompilerParams(use_tc_tiling_on_sc=False),
        scratch_types=(
            [pltpu.VMEM((CH,), jnp.int32), pltpu.VMEM((CH,), jnp.int32)] +
            [pltpu.VMEM((CH, 16), jnp.float32)] * 3 +
            [pltpu.VMEM((CH,), jnp.int32), pltpu.VMEM((CH,), jnp.int32)] +
            [pltpu.VMEM((CH, 16), jnp.float32)] * 3 +
            [pltpu.SemaphoreType.DMA] * 6
        ),
    )(a_pad, b_pad, vidx, eidx)


RB = 1000
GRID = N // RB


def _row_spec(cols):
    return pl.BlockSpec((RB, cols), lambda i: (i, 0))


def _full_spec(r, c):
    return pl.BlockSpec((r, c), lambda i: (0, 0))


def _tc_d0_body(x_ref, wl_ref, bl_ref, wsa_ref, xh_ref, ap_ref):
    xh = jax.nn.relu(jnp.dot(x_ref[...], wl_ref[...],
                             preferred_element_type=jnp.float32) + bl_ref[...])
    xh_ref[...] = xh
    ap_ref[...] = jnp.dot(xh, wsa_ref[...], preferred_element_type=jnp.float32)


def _tc_d0(x, W_lin, b_lin, Wsa):
    return pl.pallas_call(
        _tc_d0_body,
        grid=(GRID,),
        in_specs=[_row_spec(FIN), _full_spec(FIN, DH), _full_spec(1, DH),
                  _full_spec(DH, 16)],
        out_specs=[_row_spec(DH), _row_spec(16)],
        out_shape=[jax.ShapeDtypeStruct((N, DH), jnp.float32),
                   jax.ShapeDtypeStruct((N, 16), jnp.float32)],
    )(x, W_lin, b_lin, Wsa)


def _tc_d1_body(p0_ref, p1_ref, c0_ref, c1_ref, wl_ref, bl_ref, wsb_ref,
                bsh_ref, heh_ref, bp_ref):
    cnt = c0_ref[:, 0:1] + c1_ref[:, 0:1]
    he = (p0_ref[...] + p1_ref[...]) / jnp.maximum(cnt, 1.0)
    heh = jax.nn.relu(jnp.dot(he, wl_ref[...],
                              preferred_element_type=jnp.float32) + bl_ref[...])
    heh_ref[...] = heh
    bp_ref[...] = jnp.dot(heh, wsb_ref[...],
                          preferred_element_type=jnp.float32) + bsh_ref[...]


def _tc_d1(p0, p1, c0, c1, W_lin, b_lin, Wsb, bsh):
    return pl.pallas_call(
        _tc_d1_body,
        grid=(GRID,),
        in_specs=[_row_spec(FIN), _row_spec(FIN), _row_spec(16), _row_spec(16),
                  _full_spec(FIN, DH), _full_spec(1, DH), _full_spec(DH, 16),
                  _full_spec(1, 16)],
        out_specs=[_row_spec(DH), _row_spec(16)],
        out_shape=[jax.ShapeDtypeStruct((NE, DH), jnp.float32),
                   jax.ShapeDtypeStruct((NE, 16), jnp.float32)],
    )(p0, p1, c0, c1, W_lin, b_lin, Wsb, bsh)


def _tc_mm_body(x_ref, w_ref, b_ref, o_ref):
    o_ref[...] = jnp.dot(x_ref[...], w_ref[...],
                         preferred_element_type=jnp.float32) + b_ref[...]


def _tc_mm(x, w, b):
    k, m = w.shape
    return pl.pallas_call(
        _tc_mm_body,
        grid=(GRID,),
        in_specs=[_row_spec(k), _full_spec(k, m), _full_spec(1, m)],
        out_specs=_row_spec(m),
        out_shape=jax.ShapeDtypeStruct((x.shape[0], m), jnp.float32),
    )(x, w, b)


def _tc_d3_body(p0_ref, p1_ref, w_ref, o_ref):
    o_ref[...] = jnp.dot(p0_ref[...] + p1_ref[...], w_ref[...],
                         preferred_element_type=jnp.float32)


def _tc_d3(p0, p1, w):
    return pl.pallas_call(
        _tc_d3_body,
        grid=(GRID,),
        in_specs=[_row_spec(DH), _row_spec(DH), _full_spec(DH, DH)],
        out_specs=_row_spec(DH),
        out_shape=jax.ShapeDtypeStruct((NE, DH), jnp.float32),
    )(p0, p1, w)


def _tc_d4_body(s0_ref, s1_ref, a2_ref, sp0_ref, sp1_ref, x0_ref,
                w3_ref, b3_ref, b2_ref, o_ref):
    s2 = sp0_ref[:, 0:2] + sp1_ref[:, 0:2]
    sexp = jnp.concatenate(
        [jnp.broadcast_to(s2[:, 0:1], (RB, H)),
         jnp.broadcast_to(s2[:, 1:2], (RB, H))], axis=1)
    xv = (s0_ref[...] + s1_ref[...]) + (a2_ref[...] + b2_ref[...]) * sexp
    z = (1.0 - ALPHA) * xv + ALPHA * x0_ref[...]
    o_ref[...] = jax.nn.relu(jnp.dot(z, w3_ref[...],
                                     preferred_element_type=jnp.float32)
                             + b3_ref[...])


def _tc_d4(S0, S1, A2, sp0, sp1, x0, W3b, b3t, b2t):
    return pl.pallas_call(
        _tc_d4_body,
        grid=(GRID,),
        in_specs=[_row_spec(DH), _row_spec(DH), _row_spec(DH), _row_spec(16),
                  _row_spec(16), _row_spec(DH), _full_spec(DH, DH),
                  _full_spec(1, DH), _full_spec(1, DH)],
        out_specs=_row_spec(DH),
        out_shape=jax.ShapeDtypeStruct((N, DH), jnp.float32),
    )(S0, S1, A2, sp0, sp1, x0, W3b, b3t, b2t)


def _tc_cls_body(x_ref, w1_ref, b1_ref, w2_ref, b2_ref, o_ref):
    h = jax.nn.relu(jnp.dot(x_ref[...], w1_ref[...],
                            preferred_element_type=jnp.float32) + b1_ref[...])
    o_ref[...] = jnp.dot(h, w2_ref[...],
                         preferred_element_type=jnp.float32) + b2_ref[...]


def _tc_cls(x, Wc1, bc1, Wc2, bc2):
    return pl.pallas_call(
        _tc_cls_body,
        grid=(GRID,),
        in_specs=[_row_spec(DH), _full_spec(DH, 64), _full_spec(1, 64),
                  _full_spec(64, NCLS), _full_spec(1, NCLS)],
        out_specs=_row_spec(NCLS),
        out_shape=jax.ShapeDtypeStruct((N, NCLS), jnp.float32),
    )(x, Wc1, bc1, Wc2, bc2)


def kernel(x, edge_index, W_lin, b_lin, W_sheaf, b_sheaf, W1, b1, W2, b2,
           W3, b3, Wc1, bc1, Wc2, bc2):
    V = edge_index[0].astype(jnp.int32)
    E = edge_index[1].astype(jnp.int32)

    b_lin2 = b_lin[None, :]
    bc1_2 = bc1[None, :]
    bc2_2 = bc2[None, :]
    eye2 = jnp.eye(2, dtype=jnp.float32)
    W1b = jnp.kron(eye2, W1)
    W2a = jnp.kron(eye2, W2[:H])
    W2b = jnp.kron(eye2, W2[H:])
    W3b = jnp.kron(eye2, W3)
    b1t = jnp.tile(b1, 2)[None, :]
    b2t = jnp.tile(b2, 2)[None, :]
    b3t = jnp.tile(b3, 2)[None, :]
    Wsa = jnp.pad(W_sheaf[:DH], ((0, 0), (0, 16 - D)))
    Wsb = jnp.pad(W_sheaf[DH:], ((0, 0), (0, 16 - D)))
    bsh = jnp.pad(b_sheaf, (0, 16 - D))[None, :]
    Wcat = jnp.concatenate([W1b, W2a], axis=1)
    bcat = jnp.concatenate([b1t, jnp.zeros_like(b1t)], axis=1)

    z128 = jnp.zeros((RZ, FIN), jnp.float32)
    z16 = jnp.zeros((RZ, 16), jnp.float32)
    ones_hd = jnp.ones((NNZ, 16), jnp.float32)
    flag0 = jnp.zeros((16,), jnp.int32)
    flag1 = jnp.ones((16,), jnp.int32)
    flag2 = jnp.full((16,), 2, jnp.int32)

    xh, a_pad = _tc_d0(x, W_lin, b_lin2, Wsa)
    hep, cntp, _u0 = _sc_scaled(x, V, E, ones_hd, a_pad, a_pad, flag0,
                                z128, z16)
    heh, b_pad = _tc_d1(hep[0], hep[1], cntp[0], cntp[1],
                        W_lin, b_lin2, Wsb, bsh)

    x0_2 = xh
    Xc2 = xh
    hd = ones_hd
    sp = cntp
    for layer in range(2):
        pa = _tc_mm(Xc2, Wcat, bcat)
        P2 = pa[:, :DH]
        A2 = pa[:, DH:]
        if layer == 0:
            hd = _sc_sheaf(a_pad, b_pad, V, E)
        xep, _u1, _u2 = _sc_scaled(P2, V, E, hd, a_pad, b_pad, flag1,
                                   z128, z16)
        Q2 = _tc_d3(xep[0], xep[1], W2b)
        s2p, sp, _u3 = _sc_scaled(Q2, E, V, hd, a_pad, b_pad, flag1,
                                  z128, z16)
        Xc2 = _tc_d4(s2p[0], s2p[1], A2, sp[0], sp[1], x0_2, W3b, b3t, b2t)

    return _tc_cls(Xc2, Wc1, bc1_2, Wc2, bc2_2)


# speedup vs baseline: 1.0841x; 1.0841x over previous
import jax
import jax.numpy as jnp
from jax import lax
from jax.experimental import pallas as pl
from jax.experimental.pallas import tpu as pltpu
from jax.experimental.pallas import tpu_sc as plsc

N = 10000
NNZ = 320000
FIN = 128
D = 2
H = 64
DH = D * H
NCLS = 40
NE = 10000
ALPHA = 0.5

NCORE = 2
NSUB = 16
NWORK = NCORE * NSUB
PS = NNZ // NWORK
CH = 80
NCHUNK = PS // CH
RZ = 1000
NZSUB = N // RZ

_SC_MESH = dict(core_axis_name="c", subcore_axis_name="s")


def _wid():
    return lax.axis_index("s") * NCORE + lax.axis_index("c")


def _zero_acc(z_hbm, acc, sid):
    @pl.when(sid < NZSUB)
    def _():
        pltpu.sync_copy(z_hbm, acc.at[pl.ds(sid * RZ, RZ)])


def _writeout(acc, out_hbm, cid, sid):
    @pl.when(sid < NZSUB)
    def _():
        pltpu.sync_copy(acc.at[pl.ds(sid * RZ, RZ)],
                        out_hbm.at[cid, pl.ds(sid * RZ, RZ)])


def _sc_scaled_body(t_hbm, s_hbm, d_hbm, hd_hbm, a_hbm, b_hbm, flag_hbm,
                    z128, z16, out_hbm, cnt_hbm, hdo_hbm,
                    sidx0, didx0, hdbuf0, sidx1, didx1, hdbuf1,
                    sidx2, didx2, hdbuf2, sidx3, didx3, hdbuf3,
                    rows0, rows1, arow0, brow0, arow1, brow1, flagbuf,
                    seml0, seml1, seml2, seml3, semg0, semg1, sema0, sema1,
                    semw0, semw1, acc, cacc):
    cid = lax.axis_index("c")
    sid = lax.axis_index("s")
    wid = _wid()
    _zero_acc(z128, acc, sid)

    @pl.when(sid < NZSUB)
    def _():
        pltpu.sync_copy(z16, cacc.at[pl.ds(sid * RZ, RZ)])
    plsc.subcore_barrier()

    base = wid * PS
    pltpu.sync_copy(flag_hbm, flagbuf)
    mode = flagbuf[pl.ds(0, 16)][0]
    do_scale = mode >= 1
    fused = mode == 2

    sets = ((sidx0, didx0, hdbuf0, seml0), (sidx1, didx1, hdbuf1, seml1),
            (sidx2, didx2, hdbuf2, seml2), (sidx3, didx3, hdbuf3, seml3))
    rbufs = ((rows0, semg0, sema0, arow0, brow0, semw0),
             (rows1, semg1, sema1, arow1, brow1, semw1))
    lane0 = jnp.zeros((16, 1), jnp.int32)
    lane1 = jnp.ones((16, 1), jnp.int32)
    gdn = lax.GatherDimensionNumbers(offset_dims=(), collapsed_slice_dims=(0,),
                                     start_index_map=(0,))

    def bcast(hv, lanes):
        return lax.gather(hv, lanes, gdn, (1,),
                          mode=lax.GatherScatterMode.PROMISE_IN_BOUNDS)

    def issue_linear(j, q):
        off = base + j * CH
        pltpu.async_copy(s_hbm.at[pl.ds(off, CH)], sets[q][0], sets[q][3])
        pltpu.async_copy(d_hbm.at[pl.ds(off, CH)], sets[q][1], sets[q][3])

        @pl.when(jnp.logical_not(fused))
        def _():
            pltpu.async_copy(hd_hbm.at[pl.ds(off, CH)], sets[q][2], sets[q][3])

    def wait_linear(q):
        pltpu.make_async_copy(s_hbm.at[pl.ds(0, CH)], sets[q][0], sets[q][3]).wait()
        pltpu.make_async_copy(d_hbm.at[pl.ds(0, CH)], sets[q][1], sets[q][3]).wait()

        @pl.when(jnp.logical_not(fused))
        def _():
            pltpu.make_async_copy(hd_hbm.at[pl.ds(0, CH)], sets[q][2],
                                  sets[q][3]).wait()

    def issue_gathers(r, q):
        pltpu.async_copy(t_hbm.at[sets[q][0]], rbufs[r][0], rbufs[r][1])

        @pl.when(fused)
        def _():
            pltpu.async_copy(a_hbm.at[sets[q][0]], rbufs[r][3], rbufs[r][1])
            pltpu.async_copy(b_hbm.at[sets[q][1]], rbufs[r][4], rbufs[r][1])

    def wait_gathers(r, q):
        pltpu.make_async_copy(t_hbm.at[sets[q][0]], rbufs[r][0],
                              rbufs[r][1]).wait()

        @pl.when(fused)
        def _():
            pltpu.make_async_copy(a_hbm.at[sets[q][0]], rbufs[r][3],
                                  rbufs[r][1]).wait()
            pltpu.make_async_copy(b_hbm.at[sets[q][1]], rbufs[r][4],
                                  rbufs[r][1]).wait()

    def issue_scatters(r, q, j):
        pltpu.async_copy(rbufs[r][0], acc.at[sets[q][1]], rbufs[r][2],
                         add=True)
        pltpu.async_copy(sets[q][2], cacc.at[sets[q][1]], rbufs[r][2],
                         add=True)

        @pl.when(fused)
        def _():
            off = base + j * CH
            pltpu.async_copy(sets[q][2], hdo_hbm.at[pl.ds(off, CH)],
                             rbufs[r][5])

    def wait_scatter(r, q):
        pltpu.make_async_copy(rbufs[r][0], acc.at[sets[q][1]],
                              rbufs[r][2]).wait()
        pltpu.make_async_copy(sets[q][2], cacc.at[sets[q][1]],
                              rbufs[r][2]).wait()

        @pl.when(fused)
        def _():
            pltpu.make_async_copy(sets[q][2], hdo_hbm.at[pl.ds(0, CH)],
                                  rbufs[r][5]).wait()

    def compute(r, q):
        rows, hdbuf = rbufs[r][0], sets[q][2]
        arow, brow = rbufs[r][3], rbufs[r][4]

        @pl.when(fused)
        def _():
            @plsc.parallel_loop(0, CH, 1, unroll=2)
            def _(i):
                y = arow[i, :] + brow[i, :]
                u = jnp.exp(jnp.float32(-2.0) * jnp.abs(y))
                t = jnp.sign(y) * (jnp.float32(1.0) - u) / (jnp.float32(1.0) + u)
                hdbuf[i, :] = t

        @pl.when(do_scale)
        def _():
            @plsc.parallel_loop(0, CH, 1, unroll=2)
            def _(i):
                hv = hdbuf[i, :]
                s0 = bcast(hv, lane0)
                s1 = bcast(hv, lane1)
                for k in range(8):
                    sv = s0 if k < 4 else s1
                    rows[i, pl.ds(k * 16, 16)] = rows[i, pl.ds(k * 16, 16)] * sv

    issue_linear(0, 0)
    wait_linear(0)
    issue_gathers(0, 0)
    issue_linear(1, 1)

    def step(jj, _):
        for t in range(4):
            r, nr, q = t % 2, 1 - t % 2, t
            j = jj * 4 + t
            wait_gathers(r, q)
            @pl.when(j >= 1)
            def _():
                wait_scatter(nr, (t + 3) % 4)
            wait_linear((t + 1) % 4)
            issue_gathers(nr, (t + 1) % 4)
            @pl.when(j + 2 < NCHUNK)
            def _():
                issue_linear(j + 2, (t + 2) % 4)
            compute(r, q)
            issue_scatters(r, q, j)
        return 0
    lax.fori_loop(0, NCHUNK // 4, step, 0)

    wait_scatter(1, 3)
    wait_gathers(0, 0)
    compute(0, 0)
    pltpu.sync_copy(rbufs[0][0], acc.at[sets[0][1]], add=True)
    pltpu.sync_copy(sets[0][2], cacc.at[sets[0][1]], add=True)

    @pl.when(fused)
    def _():
        off = base + (NCHUNK - 1) * CH
        pltpu.sync_copy(sets[0][2], hdo_hbm.at[pl.ds(off, CH)])

    plsc.subcore_barrier()
    _writeout(acc, out_hbm, cid, sid)

    @pl.when(sid < NZSUB)
    def _():
        pltpu.sync_copy(cacc.at[pl.ds(sid * RZ, RZ)],
                        cnt_hbm.at[cid, pl.ds(sid * RZ, RZ)])


def _sc_scaled(table, src, dst, hd, a_pad, b_pad, flag, z128, z16):
    return pl.kernel(
        _sc_scaled_body,
        out_type=[jax.ShapeDtypeStruct((NCORE, N, FIN), jnp.float32),
                  jax.ShapeDtypeStruct((NCORE, N, 16), jnp.float32),
                  jax.ShapeDtypeStruct((NNZ, 16), jnp.float32)],
        mesh=plsc.VectorSubcoreMesh(**_SC_MESH),
        compiler_params=pltpu.CompilerParams(use_tc_tiling_on_sc=False),
        scratch_types=(
            [pltpu.VMEM((CH,), jnp.int32), pltpu.VMEM((CH,), jnp.int32),
             pltpu.VMEM((CH, 16), jnp.float32)] * 4 +
            [pltpu.VMEM((CH, FIN), jnp.float32)] * 2 +
            [pltpu.VMEM((CH, 16), jnp.float32)] * 4 +
            [pltpu.VMEM((16,), jnp.int32)] +
            [pltpu.SemaphoreType.DMA] * 10 +
            [pltpu.VMEM_SHARED((N, FIN), jnp.float32),
             pltpu.VMEM_SHARED((N, 16), jnp.float32)]
        ),
    )(table, src, dst, hd, a_pad, b_pad, flag, z128, z16)


RB = 1000
GRID = N // RB


def _row_spec(cols):
    return pl.BlockSpec((RB, cols), lambda i: (i, 0))


def _full_spec(r, c):
    return pl.BlockSpec((r, c), lambda i: (0, 0))


def _tc_d0_body(x_ref, wl_ref, bl_ref, wsa_ref, xh_ref, ap_ref):
    xh = jax.nn.relu(jnp.dot(x_ref[...], wl_ref[...],
                             preferred_element_type=jnp.float32) + bl_ref[...])
    xh_ref[...] = xh
    ap_ref[...] = jnp.dot(xh, wsa_ref[...], preferred_element_type=jnp.float32)


def _tc_d0(x, W_lin, b_lin, Wsa):
    return pl.pallas_call(
        _tc_d0_body,
        grid=(GRID,),
        in_specs=[_row_spec(FIN), _full_spec(FIN, DH), _full_spec(1, DH),
                  _full_spec(DH, 16)],
        out_specs=[_row_spec(DH), _row_spec(16)],
        out_shape=[jax.ShapeDtypeStruct((N, DH), jnp.float32),
                   jax.ShapeDtypeStruct((N, 16), jnp.float32)],
    )(x, W_lin, b_lin, Wsa)


def _tc_d1_body(p0_ref, p1_ref, c0_ref, c1_ref, wl_ref, bl_ref, wsb_ref,
                bsh_ref, heh_ref, bp_ref):
    cnt = c0_ref[:, 0:1] + c1_ref[:, 0:1]
    he = (p0_ref[...] + p1_ref[...]) / jnp.maximum(cnt, 1.0)
    heh = jax.nn.relu(jnp.dot(he, wl_ref[...],
                              preferred_element_type=jnp.float32) + bl_ref[...])
    heh_ref[...] = heh
    bp_ref[...] = jnp.dot(heh, wsb_ref[...],
                          preferred_element_type=jnp.float32) + bsh_ref[...]


def _tc_d1(p0, p1, c0, c1, W_lin, b_lin, Wsb, bsh):
    return pl.pallas_call(
        _tc_d1_body,
        grid=(GRID,),
        in_specs=[_row_spec(FIN), _row_spec(FIN), _row_spec(16), _row_spec(16),
                  _full_spec(FIN, DH), _full_spec(1, DH), _full_spec(DH, 16),
                  _full_spec(1, 16)],
        out_specs=[_row_spec(DH), _row_spec(16)],
        out_shape=[jax.ShapeDtypeStruct((NE, DH), jnp.float32),
                   jax.ShapeDtypeStruct((NE, 16), jnp.float32)],
    )(p0, p1, c0, c1, W_lin, b_lin, Wsb, bsh)


def _tc_mm_body(x_ref, w_ref, b_ref, o_ref):
    o_ref[...] = jnp.dot(x_ref[...], w_ref[...],
                         preferred_element_type=jnp.float32) + b_ref[...]


def _tc_mm(x, w, b):
    k, m = w.shape
    return pl.pallas_call(
        _tc_mm_body,
        grid=(GRID,),
        in_specs=[_row_spec(k), _full_spec(k, m), _full_spec(1, m)],
        out_specs=_row_spec(m),
        out_shape=jax.ShapeDtypeStruct((x.shape[0], m), jnp.float32),
    )(x, w, b)


def _tc_d3_body(p0_ref, p1_ref, w_ref, o_ref):
    o_ref[...] = jnp.dot(p0_ref[...] + p1_ref[...], w_ref[...],
                         preferred_element_type=jnp.float32)


def _tc_d3(p0, p1, w):
    return pl.pallas_call(
        _tc_d3_body,
        grid=(GRID,),
        in_specs=[_row_spec(DH), _row_spec(DH), _full_spec(DH, DH)],
        out_specs=_row_spec(DH),
        out_shape=jax.ShapeDtypeStruct((NE, DH), jnp.float32),
    )(p0, p1, w)


def _tc_d4_body(s0_ref, s1_ref, a2_ref, sp0_ref, sp1_ref, x0_ref,
                w3_ref, b3_ref, b2_ref, o_ref):
    s2 = sp0_ref[:, 0:2] + sp1_ref[:, 0:2]
    sexp = jnp.concatenate(
        [jnp.broadcast_to(s2[:, 0:1], (RB, H)),
         jnp.broadcast_to(s2[:, 1:2], (RB, H))], axis=1)
    xv = (s0_ref[...] + s1_ref[...]) + (a2_ref[...] + b2_ref[...]) * sexp
    z = (1.0 - ALPHA) * xv + ALPHA * x0_ref[...]
    o_ref[...] = jax.nn.relu(jnp.dot(z, w3_ref[...],
                                     preferred_element_type=jnp.float32)
                             + b3_ref[...])


def _tc_d4(S0, S1, A2, sp0, sp1, x0, W3b, b3t, b2t):
    return pl.pallas_call(
        _tc_d4_body,
        grid=(GRID,),
        in_specs=[_row_spec(DH), _row_spec(DH), _row_spec(DH), _row_spec(16),
                  _row_spec(16), _row_spec(DH), _full_spec(DH, DH),
                  _full_spec(1, DH), _full_spec(1, DH)],
        out_specs=_row_spec(DH),
        out_shape=jax.ShapeDtypeStruct((N, DH), jnp.float32),
    )(S0, S1, A2, sp0, sp1, x0, W3b, b3t, b2t)


def _tc_cls_body(x_ref, w1_ref, b1_ref, w2_ref, b2_ref, o_ref):
    h = jax.nn.relu(jnp.dot(x_ref[...], w1_ref[...],
                            preferred_element_type=jnp.float32) + b1_ref[...])
    o_ref[...] = jnp.dot(h, w2_ref[...],
                         preferred_element_type=jnp.float32) + b2_ref[...]


def _tc_cls(x, Wc1, bc1, Wc2, bc2):
    return pl.pallas_call(
        _tc_cls_body,
        grid=(GRID,),
        in_specs=[_row_spec(DH), _full_spec(DH, 64), _full_spec(1, 64),
                  _full_spec(64, NCLS), _full_spec(1, NCLS)],
        out_specs=_row_spec(NCLS),
        out_shape=jax.ShapeDtypeStruct((N, NCLS), jnp.float32),
    )(x, Wc1, bc1, Wc2, bc2)


def kernel(x, edge_index, W_lin, b_lin, W_sheaf, b_sheaf, W1, b1, W2, b2,
           W3, b3, Wc1, bc1, Wc2, bc2):
    V = edge_index[0].astype(jnp.int32)
    E = edge_index[1].astype(jnp.int32)

    b_lin2 = b_lin[None, :]
    bc1_2 = bc1[None, :]
    bc2_2 = bc2[None, :]
    eye2 = jnp.eye(2, dtype=jnp.float32)
    W1b = jnp.kron(eye2, W1)
    W2a = jnp.kron(eye2, W2[:H])
    W2b = jnp.kron(eye2, W2[H:])
    W3b = jnp.kron(eye2, W3)
    b1t = jnp.tile(b1, 2)[None, :]
    b2t = jnp.tile(b2, 2)[None, :]
    b3t = jnp.tile(b3, 2)[None, :]
    Wsa = jnp.pad(W_sheaf[:DH], ((0, 0), (0, 16 - D)))
    Wsb = jnp.pad(W_sheaf[DH:], ((0, 0), (0, 16 - D)))
    bsh = jnp.pad(b_sheaf, (0, 16 - D))[None, :]
    Wcat = jnp.concatenate([W1b, W2a], axis=1)
    bcat = jnp.concatenate([b1t, jnp.zeros_like(b1t)], axis=1)

    z128 = jnp.zeros((RZ, FIN), jnp.float32)
    z16 = jnp.zeros((RZ, 16), jnp.float32)
    ones_hd = jnp.ones((NNZ, 16), jnp.float32)
    flag0 = jnp.zeros((16,), jnp.int32)
    flag1 = jnp.ones((16,), jnp.int32)
    flag2 = jnp.full((16,), 2, jnp.int32)

    xh, a_pad = _tc_d0(x, W_lin, b_lin2, Wsa)
    hep, cntp, _u0 = _sc_scaled(x, V, E, ones_hd, a_pad, a_pad, flag0,
                                z128, z16)
    heh, b_pad = _tc_d1(hep[0], hep[1], cntp[0], cntp[1],
                        W_lin, b_lin2, Wsb, bsh)

    x0_2 = xh
    Xc2 = xh
    hd = ones_hd
    sp = cntp
    for layer in range(2):
        pa = _tc_mm(Xc2, Wcat, bcat)
        P2 = pa[:, :DH]
        A2 = pa[:, DH:]
        if layer == 0:
            xep, _u1, hd = _sc_scaled(P2, V, E, ones_hd, a_pad, b_pad,
                                      flag2, z128, z16)
        else:
            xep, _u1, _u2 = _sc_scaled(P2, V, E, hd, a_pad, b_pad, flag1,
                                       z128, z16)
        Q2 = _tc_d3(xep[0], xep[1], W2b)
        s2p, sp, _u3 = _sc_scaled(Q2, E, V, hd, a_pad, b_pad, flag1,
                                  z128, z16)
        Xc2 = _tc_d4(s2p[0], s2p[1], A2, sp[0], sp[1], x0_2, W3b, b3t, b2t)

    return _tc_cls(Xc2, Wc1, bc1_2, Wc2, bc2_2)
